# baseline (speedup 1.0000x reference)
import functools

import jax
import jax.numpy as jnp
from jax.experimental import pallas as pl
from jax.experimental.pallas import tpu as pltpu

K = 20
N_BLOCKS = 7
N_FILTERS = 16
B = 4
N = 4096

def _knn_body(xsqc_ref, x_ref, xt_ref, xsqt_ref, o_ref, d_ref, *, m, dil, n):
    x_blk = x_ref[0]
    xt = xt_ref[0]
    xsqc = xsqc_ref[0]
    xsqt = xsqt_ref[0]
    inner = -2.0 * jax.lax.dot_general(
        x_blk, xt, (((1,), (0,)), ((), ())),
        preferred_element_type=jnp.float32)
    d_ref[...] = (xsqc + inner) + xsqt
    iota = jax.lax.broadcasted_iota(jnp.int32, (x_blk.shape[0], n), 1)
    out_iota = jax.lax.broadcasted_iota(jnp.int32, (x_blk.shape[0], K), 1)
    acc0 = jnp.zeros((x_blk.shape[0], K), jnp.int32)

    am0 = jnp.full((x_blk.shape[0], 1), -1, jnp.int32)

    def body(j, carry):
        acc, am_prev = carry
        d = jnp.where(iota == am_prev, 3.0e38, d_ref[...])
        d_ref[...] = d
        v = jnp.min(d, axis=1, keepdims=True)
        am = jnp.min(jnp.where(d == v, iota, n), axis=1, keepdims=True)
        keep = (j % dil) == 0
        pos = j // dil
        acc = acc + jnp.where(keep & (out_iota == pos), am, 0)
        return acc, am

    acc, _ = jax.lax.fori_loop(0, m, body, (acc0, am0))
    o_ref[0] = acc


@functools.partial(jax.jit, static_argnames=("m", "dil", "blk"))
def _knn_pallas(xt_bnc, m, dil, blk=256):
    b, n, c = xt_bnc.shape
    xsq = jnp.sum(xt_bnc * xt_bnc, axis=-1, keepdims=True)
    xsqt = jnp.swapaxes(xsq, 2, 1)
    xt_cbn = jnp.swapaxes(xt_bnc, 2, 1)
    grid = (b, n // blk)
    return pl.pallas_call(
        functools.partial(_knn_body, m=m, dil=dil, n=n),
        grid=grid,
        in_specs=[
            pl.BlockSpec((1, blk, 1), lambda i, r: (i, r, 0)),
            pl.BlockSpec((1, blk, c), lambda i, r: (i, r, 0)),
            pl.BlockSpec((1, c, n), lambda i, r: (i, 0, 0)),
            pl.BlockSpec((1, 1, n), lambda i, r: (i, 0, 0)),
        ],
        out_specs=pl.BlockSpec((1, blk, K), lambda i, r: (i, r, 0)),
        out_shape=jax.ShapeDtypeStruct((b, n, K), jnp.int32),
        scratch_shapes=[pltpu.VMEM((blk, n), jnp.float32)],
        compiler_params=pltpu.CompilerParams(
            dimension_semantics=("parallel", "parallel")),
    )(xsq, xt_bnc, xt_cbn, xsqt)


def _pad_channels(xt_bnc):
    c = xt_bnc.shape[-1]
    cp = ((c + 7) // 8) * 8
    if cp == c:
        return xt_bnc
    pad = jnp.zeros(xt_bnc.shape[:-1] + (cp - c,), xt_bnc.dtype)
    return jnp.concatenate([xt_bnc, pad], axis=-1)


def _knn_idx(x_bc_n1, m, dil):
    xt = jnp.squeeze(x_bc_n1, -1).transpose(0, 2, 1)
    return _knn_pallas(_pad_channels(xt), m, dil)


def _index_select(x, idx):
    x_sq = jnp.squeeze(x, -1)
    return jax.vmap(lambda xb, ib: xb[:, ib])(x_sq, idx)


def _basic_conv(x, W, bb, gamma, beta):
    y = jnp.einsum('oc,bcnk->bonk', W, x) + bb[None, :, None, None]
    mean = jnp.mean(y, axis=(0, 2, 3), keepdims=True)
    var = jnp.var(y, axis=(0, 2, 3), keepdims=True)
    y = (y - mean) / jnp.sqrt(var + 1e-5)
    y = y * gamma[None, :, None, None] + beta[None, :, None, None]
    return jax.nn.relu(y)


def _edge_conv(x, nn_idx, W, bb, gamma, beta):
    b, _, n, _ = x.shape
    center = jnp.broadcast_to(
        jnp.arange(n, dtype=nn_idx.dtype)[None, :, None], nn_idx.shape)
    x_i = _index_select(x, center)
    x_j = _index_select(x, nn_idx)
    out = _basic_conv(
        jnp.concatenate([x_i, x_j - x_i], axis=1), W, bb, gamma, beta)
    return jnp.max(out, axis=-1, keepdims=True)


def kernel(inputs, W_head, b_head, g_head, be_head, W_blocks, b_blocks,
           g_blocks, be_blocks):
    topo_list = []
    nn_idx = _knn_idx(inputs[:, 0:3], K, 1)
    topo_list.append(nn_idx)
    feat = _edge_conv(inputs, nn_idx, W_head, b_head, g_head, be_head)
    for i in range(N_BLOCKS - 1):
        nn_idx = _knn_idx(feat, K * (1 + i), 1 + i)
        out = _edge_conv(feat, nn_idx, W_blocks[i], b_blocks[i], g_blocks[i],
                         be_blocks[i])
        feat = out + feat
        topo_list.append(nn_idx)
    out_feat = jnp.swapaxes(jnp.squeeze(feat, -1), 1, 2)
    return (out_feat, jnp.stack(topo_list, axis=0))


# speedup vs baseline: 1.0187x; 1.0187x over previous
import functools

import jax
import jax.numpy as jnp
from jax.experimental import pallas as pl
from jax.experimental.pallas import tpu as pltpu

K = 20
N_BLOCKS = 7
N_FILTERS = 16
B = 4
N = 4096

def _knn_body(xsqc_ref, x_ref, xt_ref, xsqt_ref, o_ref, d_ref, *, m, dil, n):
    x_blk = x_ref[0]
    xt = xt_ref[0]
    xsqc = xsqc_ref[0]
    xsqt = xsqt_ref[0]
    inner = -2.0 * jax.lax.dot_general(
        x_blk, xt, (((1,), (0,)), ((), ())),
        preferred_element_type=jnp.float32)
    d_ref[...] = (xsqc + inner) + xsqt
    iota = jax.lax.broadcasted_iota(jnp.int32, (x_blk.shape[0], n), 1)
    out_iota = jax.lax.broadcasted_iota(jnp.int32, (x_blk.shape[0], K), 1)
    acc0 = jnp.zeros((x_blk.shape[0], K), jnp.int32)

    am0 = jnp.full((x_blk.shape[0], 1), -1, jnp.int32)

    def body(j, carry):
        acc, am_prev = carry
        d = jnp.where(iota == am_prev, 3.0e38, d_ref[...])
        d_ref[...] = d
        v = jnp.min(d, axis=1, keepdims=True)
        am = jnp.min(jnp.where(d == v, iota, n), axis=1, keepdims=True)
        keep = (j % dil) == 0
        pos = j // dil
        acc = acc + jnp.where(keep & (out_iota == pos), am, 0)
        return acc, am

    acc, _ = jax.lax.fori_loop(0, m, body, (acc0, am0))
    o_ref[0] = acc


@functools.partial(jax.jit, static_argnames=("m", "dil", "blk"))
def _knn_pallas(xt_bnc, m, dil, blk=512):
    b, n, c = xt_bnc.shape
    xsq = jnp.sum(xt_bnc * xt_bnc, axis=-1, keepdims=True)
    xsqt = jnp.swapaxes(xsq, 2, 1)
    xt_cbn = jnp.swapaxes(xt_bnc, 2, 1)
    grid = (b, n // blk)
    return pl.pallas_call(
        functools.partial(_knn_body, m=m, dil=dil, n=n),
        grid=grid,
        in_specs=[
            pl.BlockSpec((1, blk, 1), lambda i, r: (i, r, 0)),
            pl.BlockSpec((1, blk, c), lambda i, r: (i, r, 0)),
            pl.BlockSpec((1, c, n), lambda i, r: (i, 0, 0)),
            pl.BlockSpec((1, 1, n), lambda i, r: (i, 0, 0)),
        ],
        out_specs=pl.BlockSpec((1, blk, K), lambda i, r: (i, r, 0)),
        out_shape=jax.ShapeDtypeStruct((b, n, K), jnp.int32),
        scratch_shapes=[pltpu.VMEM((blk, n), jnp.float32)],
        compiler_params=pltpu.CompilerParams(
            dimension_semantics=("parallel", "parallel")),
    )(xsq, xt_bnc, xt_cbn, xsqt)


def _pad_channels(xt_bnc):
    c = xt_bnc.shape[-1]
    cp = ((c + 7) // 8) * 8
    if cp == c:
        return xt_bnc
    pad = jnp.zeros(xt_bnc.shape[:-1] + (cp - c,), xt_bnc.dtype)
    return jnp.concatenate([xt_bnc, pad], axis=-1)


def _knn_idx(x_bc_n1, m, dil):
    xt = jnp.squeeze(x_bc_n1, -1).transpose(0, 2, 1)
    return _knn_pallas(_pad_channels(xt), m, dil)


def _index_select(x, idx):
    x_sq = jnp.squeeze(x, -1)
    return jax.vmap(lambda xb, ib: xb[:, ib])(x_sq, idx)


def _basic_conv(x, W, bb, gamma, beta):
    y = jnp.einsum('oc,bcnk->bonk', W, x) + bb[None, :, None, None]
    mean = jnp.mean(y, axis=(0, 2, 3), keepdims=True)
    var = jnp.var(y, axis=(0, 2, 3), keepdims=True)
    y = (y - mean) / jnp.sqrt(var + 1e-5)
    y = y * gamma[None, :, None, None] + beta[None, :, None, None]
    return jax.nn.relu(y)


def _edge_conv(x, nn_idx, W, bb, gamma, beta):
    b, _, n, _ = x.shape
    center = jnp.broadcast_to(
        jnp.arange(n, dtype=nn_idx.dtype)[None, :, None], nn_idx.shape)
    x_i = _index_select(x, center)
    x_j = _index_select(x, nn_idx)
    out = _basic_conv(
        jnp.concatenate([x_i, x_j - x_i], axis=1), W, bb, gamma, beta)
    return jnp.max(out, axis=-1, keepdims=True)


def kernel(inputs, W_head, b_head, g_head, be_head, W_blocks, b_blocks,
           g_blocks, be_blocks):
    topo_list = []
    nn_idx = _knn_idx(inputs[:, 0:3], K, 1)
    topo_list.append(nn_idx)
    feat = _edge_conv(inputs, nn_idx, W_head, b_head, g_head, be_head)
    for i in range(N_BLOCKS - 1):
        nn_idx = _knn_idx(feat, K * (1 + i), 1 + i)
        out = _edge_conv(feat, nn_idx, W_blocks[i], b_blocks[i], g_blocks[i],
                         be_blocks[i])
        feat = out + feat
        topo_list.append(nn_idx)
    out_feat = jnp.swapaxes(jnp.squeeze(feat, -1), 1, 2)
    return (out_feat, jnp.stack(topo_list, axis=0))


# speedup vs baseline: 1.0397x; 1.0206x over previous
import functools

import jax
import jax.numpy as jnp
from jax.experimental import pallas as pl
from jax.experimental.pallas import tpu as pltpu

K = 20
N_BLOCKS = 7
N_FILTERS = 16
B = 4
N = 4096

def _knn_body(xsqc_ref, x_ref, xt_ref, xsqt_ref, o_ref, d_ref, *, m, dil, n):
    x_blk = x_ref[0]
    xt = xt_ref[0]
    xsqc = xsqc_ref[0]
    xsqt = xsqt_ref[0]
    inner = -2.0 * jax.lax.dot_general(
        x_blk, xt, (((1,), (0,)), ((), ())),
        preferred_element_type=jnp.float32)
    d_ref[...] = (xsqc + inner) + xsqt
    iota = jax.lax.broadcasted_iota(jnp.int32, (x_blk.shape[0], n), 1)
    out_iota = jax.lax.broadcasted_iota(jnp.int32, (x_blk.shape[0], K), 1)
    acc0 = jnp.zeros((x_blk.shape[0], K), jnp.int32)

    am0 = jnp.full((x_blk.shape[0], 1), -1, jnp.int32)
    nch = n // 128

    def _tree(op, xs):
        while len(xs) > 1:
            xs = [op(xs[2 * i], xs[2 * i + 1]) for i in range(len(xs) // 2)] \
                + xs[len(xs) - len(xs) % 2:]
        return xs[0]

    def body(j, carry):
        acc, am_prev = carry
        dcs, mins = [], []
        for c in range(nch):
            sl = slice(128 * c, 128 * (c + 1))
            dc = jnp.where(iota[:, sl] == am_prev, 3.0e38, d_ref[:, sl])
            d_ref[:, sl] = dc
            dcs.append(dc)
            mins.append(dc)
        v = jnp.min(_tree(jnp.minimum, mins), axis=1, keepdims=True)
        cands = [jnp.where(dcs[c] == v, iota[:, 128 * c:128 * (c + 1)], n)
                 for c in range(nch)]
        am = jnp.min(_tree(jnp.minimum, cands), axis=1, keepdims=True)
        keep = (j % dil) == 0
        pos = j // dil
        acc = acc + jnp.where(keep & (out_iota == pos), am, 0)
        return acc, am

    acc, _ = jax.lax.fori_loop(0, m, body, (acc0, am0))
    o_ref[0] = acc


@functools.partial(jax.jit, static_argnames=("m", "dil", "blk"))
def _knn_pallas(xt_bnc, m, dil, blk=512):
    b, n, c = xt_bnc.shape
    xsq = jnp.sum(xt_bnc * xt_bnc, axis=-1, keepdims=True)
    xsqt = jnp.swapaxes(xsq, 2, 1)
    xt_cbn = jnp.swapaxes(xt_bnc, 2, 1)
    grid = (b, n // blk)
    return pl.pallas_call(
        functools.partial(_knn_body, m=m, dil=dil, n=n),
        grid=grid,
        in_specs=[
            pl.BlockSpec((1, blk, 1), lambda i, r: (i, r, 0)),
            pl.BlockSpec((1, blk, c), lambda i, r: (i, r, 0)),
            pl.BlockSpec((1, c, n), lambda i, r: (i, 0, 0)),
            pl.BlockSpec((1, 1, n), lambda i, r: (i, 0, 0)),
        ],
        out_specs=pl.BlockSpec((1, blk, K), lambda i, r: (i, r, 0)),
        out_shape=jax.ShapeDtypeStruct((b, n, K), jnp.int32),
        scratch_shapes=[pltpu.VMEM((blk, n), jnp.float32)],
        compiler_params=pltpu.CompilerParams(
            dimension_semantics=("parallel", "parallel")),
    )(xsq, xt_bnc, xt_cbn, xsqt)


def _pad_channels(xt_bnc):
    c = xt_bnc.shape[-1]
    cp = ((c + 7) // 8) * 8
    if cp == c:
        return xt_bnc
    pad = jnp.zeros(xt_bnc.shape[:-1] + (cp - c,), xt_bnc.dtype)
    return jnp.concatenate([xt_bnc, pad], axis=-1)


def _knn_idx(x_bc_n1, m, dil):
    xt = jnp.squeeze(x_bc_n1, -1).transpose(0, 2, 1)
    return _knn_pallas(_pad_channels(xt), m, dil)


def _index_select(x, idx):
    x_sq = jnp.squeeze(x, -1)
    return jax.vmap(lambda xb, ib: xb[:, ib])(x_sq, idx)


def _basic_conv(x, W, bb, gamma, beta):
    y = jnp.einsum('oc,bcnk->bonk', W, x) + bb[None, :, None, None]
    mean = jnp.mean(y, axis=(0, 2, 3), keepdims=True)
    var = jnp.var(y, axis=(0, 2, 3), keepdims=True)
    y = (y - mean) / jnp.sqrt(var + 1e-5)
    y = y * gamma[None, :, None, None] + beta[None, :, None, None]
    return jax.nn.relu(y)


def _edge_conv(x, nn_idx, W, bb, gamma, beta):
    b, _, n, _ = x.shape
    center = jnp.broadcast_to(
        jnp.arange(n, dtype=nn_idx.dtype)[None, :, None], nn_idx.shape)
    x_i = _index_select(x, center)
    x_j = _index_select(x, nn_idx)
    out = _basic_conv(
        jnp.concatenate([x_i, x_j - x_i], axis=1), W, bb, gamma, beta)
    return jnp.max(out, axis=-1, keepdims=True)


def kernel(inputs, W_head, b_head, g_head, be_head, W_blocks, b_blocks,
           g_blocks, be_blocks):
    topo_list = []
    nn_idx = _knn_idx(inputs[:, 0:3], K, 1)
    topo_list.append(nn_idx)
    feat = _edge_conv(inputs, nn_idx, W_head, b_head, g_head, be_head)
    for i in range(N_BLOCKS - 1):
        nn_idx = _knn_idx(feat, K * (1 + i), 1 + i)
        out = _edge_conv(feat, nn_idx, W_blocks[i], b_blocks[i], g_blocks[i],
                         be_blocks[i])
        feat = out + feat
        topo_list.append(nn_idx)
    out_feat = jnp.swapaxes(jnp.squeeze(feat, -1), 1, 2)
    return (out_feat, jnp.stack(topo_list, axis=0))
